# baseline (speedup 1.0000x reference)
import functools
import jax
import jax.numpy as jnp
from jax import lax
from jax.experimental import pallas as pl
from jax.experimental.pallas import tpu as pltpu
from jax.experimental.pallas import tpu_sc as plsc

N = 10000
HID = 64
NUM_GRAPHS = 64
E = 320000

NC = 2
NS = 16
NW = NC * NS
G = 128
NG = 79
EPW = NG * G
EPAD = NW * EPW
NPAD = 10240
ROWS_PER_TILE = NPAD // NS


def _ka_body(x_ref, as_ref, ad_ref, w_ref, h_ref, asrc_ref, adst_ref,
             num0_ref, den0_ref):
    h = jnp.dot(x_ref[...], w_ref[...], preferred_element_type=jnp.float32)
    h_ref[...] = h
    asrc = jnp.sum(h * as_ref[...], axis=1, keepdims=True)
    adst = jnp.sum(h * ad_ref[...], axis=1, keepdims=True)
    asrc_ref[...] = asrc
    adst_ref[...] = adst
    e = asrc + adst
    e = jnp.where(e >= 0, e, 0.2 * e)
    exs = jnp.exp(e)
    num0_ref[...] = h * exs
    den0_ref[...] = exs


def _layer_prologue(x, W, a_s, a_d):
    return pl.pallas_call(
        _ka_body,
        out_shape=(
            jax.ShapeDtypeStruct((N, HID), jnp.float32),
            jax.ShapeDtypeStruct((N, 1), jnp.float32),
            jax.ShapeDtypeStruct((N, 1), jnp.float32),
            jax.ShapeDtypeStruct((N, HID), jnp.float32),
            jax.ShapeDtypeStruct((N, 1), jnp.float32),
        ),
    )(x, a_s.reshape(1, HID), a_d.reshape(1, HID), W)


def _sc_edge_body(h_hbm, asrc_hbm, adst_hbm, src_hbm, dst_hbm, znum_hbm,
                  zden_hbm, num_out, den_out,
                  src_v, dst_v, ex_v, asrc_v, adst_v, rows_v, num_sh, den_sh,
                  sem, sem_sn, sem_sd):
    c = lax.axis_index("c")
    s = lax.axis_index("s")
    wid = s * NC + c

    pltpu.sync_copy(asrc_hbm, asrc_v)
    pltpu.sync_copy(adst_hbm, adst_v)
    pltpu.sync_copy(src_hbm.at[wid], src_v)
    pltpu.sync_copy(dst_hbm.at[wid], dst_v)

    row0 = s * ROWS_PER_TILE
    pltpu.sync_copy(znum_hbm.at[pl.ds(row0, ROWS_PER_TILE)],
                    num_sh.at[pl.ds(row0, ROWS_PER_TILE)])
    pltpu.sync_copy(zden_hbm.at[pl.ds(row0, ROWS_PER_TILE)],
                    den_sh.at[pl.ds(row0, ROWS_PER_TILE)])
    plsc.subcore_barrier()

    dnums = lax.GatherDimensionNumbers(
        offset_dims=(), collapsed_slice_dims=(0,), start_index_map=(0,))

    def vec_body(k, carry):
        g, b = carry
        base = k * 16
        sidx = src_v[g, pl.ds(base, 16)]
        didx = dst_v[g, pl.ds(base, 16)]
        a = plsc.load_gather(asrc_v, [sidx])
        bb = plsc.load_gather(adst_v, [didx])
        e = a + bb
        e = jnp.where(e >= 0.0, e, 0.2 * e)
        ex16 = jnp.exp(e)
        ex_v[g, pl.ds(base, 16)] = ex16
        for j in range(16):
            exj = lax.gather(ex16, jnp.full((16, 1), j, jnp.int32), dnums,
                             (1,), mode=lax.GatherScatterMode.PROMISE_IN_BOUNDS)
            i = base + j
            for q in range(4):
                sl = pl.ds(q * 16, 16)
                rows_v[b, i, sl] = rows_v[b, i, sl] * exj
        return carry

    pltpu.async_copy(h_hbm.at[src_v.at[0]], rows_v.at[0], sem)

    def group_body(g, _):
        b = lax.rem(g, 4)
        p = lax.rem(g, 2)
        pltpu.make_async_copy(h_hbm.at[src_v.at[g]], rows_v.at[b], sem).wait()

        @pl.when(g >= 2)
        def _():
            ob = lax.rem(g + 2, 4)
            pltpu.make_async_copy(rows_v.at[ob], num_sh.at[dst_v.at[g]],
                                  sem_sn.at[p]).wait()
            pltpu.make_async_copy(ex_v.at[g], den_sh.at[dst_v.at[g]],
                                  sem_sd.at[p]).wait()

        @pl.when(g + 1 < NG)
        def _():
            nb = lax.rem(g + 1, 4)
            pltpu.async_copy(h_hbm.at[src_v.at[g + 1]], rows_v.at[nb], sem)

        lax.fori_loop(0, G // 16, vec_body, (g, b))
        pltpu.async_copy(rows_v.at[b], num_sh.at[dst_v.at[g]], sem_sn.at[p],
                         add=True)
        pltpu.async_copy(ex_v.at[g], den_sh.at[dst_v.at[g]], sem_sd.at[p],
                         add=True)
        return 0

    lax.fori_loop(0, NG, group_body, 0)
    for gg in (NG - 2, NG - 1):
        pltpu.make_async_copy(rows_v.at[gg % 4], num_sh.at[dst_v.at[gg]],
                              sem_sn.at[gg % 2]).wait()
        pltpu.make_async_copy(ex_v.at[gg], den_sh.at[dst_v.at[gg]],
                              sem_sd.at[gg % 2]).wait()
    plsc.subcore_barrier()

    pltpu.sync_copy(num_sh.at[pl.ds(row0, ROWS_PER_TILE)],
                    num_out.at[c, pl.ds(row0, ROWS_PER_TILE)])
    pltpu.sync_copy(den_sh.at[pl.ds(row0, ROWS_PER_TILE)],
                    den_out.at[pl.ds(c * NPAD + row0, ROWS_PER_TILE)])


_sc_edge_pass = pl.kernel(
    _sc_edge_body,
    out_type=(
        jax.ShapeDtypeStruct((NC, NPAD, HID), jnp.float32),
        jax.ShapeDtypeStruct((NC * NPAD,), jnp.float32),
    ),
    mesh=plsc.VectorSubcoreMesh(core_axis_name="c", subcore_axis_name="s"),
    compiler_params=pltpu.CompilerParams(needs_layout_passes=False,
                                         use_tc_tiling_on_sc=False),
    scratch_types=[
        pltpu.VMEM((NG, G), jnp.int32),
        pltpu.VMEM((NG, G), jnp.int32),
        pltpu.VMEM((NG, G), jnp.float32),
        pltpu.VMEM((N,), jnp.float32),
        pltpu.VMEM((N,), jnp.float32),
        pltpu.VMEM((4, G, HID), jnp.float32),
        pltpu.VMEM_SHARED((NPAD, HID), jnp.float32),
        pltpu.VMEM_SHARED((NPAD,), jnp.float32),
        pltpu.SemaphoreType.DMA,
        pltpu.SemaphoreType.DMA((2,)),
        pltpu.SemaphoreType.DMA((2,)),
    ],
)


def _edge_pass(h, asrc, adst, src_r, dst_r, znum, zden):
    num_p, den_p = _sc_edge_pass(
        h, asrc.reshape(N), adst.reshape(N), src_r, dst_r, znum, zden)
    return num_p, den_p


def _kb_body(num0_ref, den0_ref, nump_ref, denp_ref, b_ref, g_ref, be_ref,
             o_ref):
    num = num0_ref[...] + nump_ref[0, :N, :] + nump_ref[1, :N, :]
    den = den0_ref[...] + (denp_ref[pl.ds(0, N)]
                           + denp_ref[pl.ds(NPAD, N)]).reshape(N, 1)
    v = num / (den + 1e-16) + b_ref[...]
    v = jnp.maximum(v, 0.0)
    mu = jnp.mean(v, axis=0, keepdims=True)
    var = jnp.mean((v - mu) ** 2, axis=0, keepdims=True)
    o_ref[...] = g_ref[...] * (v - mu) / jnp.sqrt(var + 1e-5) + be_ref[...]


def _layer_epilogue(num0, den0, num_p, den_p, b, g, be):
    return pl.pallas_call(
        _kb_body,
        out_shape=jax.ShapeDtypeStruct((N, HID), jnp.float32),
    )(num0, den0, num_p, den_p, b.reshape(1, HID),
      g.reshape(1, HID), be.reshape(1, HID))


def _kc_body(h_ref, batch_ref, w1_ref, b1_ref, w2_ref, b2_ref, w3_ref, b3_ref,
             w4_ref, b4_ref, o_ref):
    gids = jax.lax.broadcasted_iota(jnp.int32, (N, NUM_GRAPHS), 1)
    onehot = (batch_ref[...] == gids).astype(jnp.float32)
    pooled = jax.lax.dot_general(
        onehot, h_ref[...], (((0,), (0,)), ((), ())),
        preferred_element_type=jnp.float32,
        precision=jax.lax.Precision.HIGHEST)
    o = jnp.dot(pooled, w1_ref[...], preferred_element_type=jnp.float32) + b1_ref[...]
    o = jnp.dot(o, w2_ref[...], preferred_element_type=jnp.float32) + b2_ref[...]
    o = jnp.dot(o, w3_ref[...], preferred_element_type=jnp.float32) + b3_ref[...]
    o = jnp.dot(o, w4_ref[...], preferred_element_type=jnp.float32) + b4_ref[...]
    o_ref[...] = o


def _head(h, batch, Wm1, bm1, Wm2, bm2, Wm3, bm3, Wm4, bm4):
    return pl.pallas_call(
        _kc_body,
        out_shape=jax.ShapeDtypeStruct((NUM_GRAPHS, 1), jnp.float32),
    )(h, batch.reshape(N, 1), Wm1, bm1.reshape(1, HID), Wm2, bm2.reshape(1, HID),
      Wm3, bm3.reshape(1, HID), Wm4, bm4.reshape(1, 1))


@jax.jit
def kernel(x, edge_index, batch, W1, a_src1, a_dst1, b1, g1, be1,
           W2, a_src2, a_dst2, b2, g2, be2,
           Wm1, bm1, Wm2, bm2, Wm3, bm3, Wm4, bm4):
    npad_e = EPAD - E
    src_r = jnp.concatenate(
        [edge_index[0], jnp.zeros((npad_e,), jnp.int32)]).reshape(NW, NG, G)
    dst_r = jnp.concatenate(
        [edge_index[1], jnp.full((npad_e,), N, jnp.int32)]).reshape(NW, NG, G)
    znum = jnp.zeros((NPAD, HID), jnp.float32)
    zden = jnp.zeros((NPAD,), jnp.float32)

    h1, asrc1, adst1, num01, den01 = _layer_prologue(x, W1, a_src1, a_dst1)
    num_p1, den_p1 = _edge_pass(h1, asrc1, adst1, src_r, dst_r, znum, zden)
    z1 = _layer_epilogue(num01, den01, num_p1, den_p1, b1, g1, be1)

    h2, asrc2, adst2, num02, den02 = _layer_prologue(z1, W2, a_src2, a_dst2)
    num_p2, den_p2 = _edge_pass(h2, asrc2, adst2, src_r, dst_r, znum, zden)
    z2 = _layer_epilogue(num02, den02, num_p2, den_p2, b2, g2, be2)

    return _head(z2, batch, Wm1, bm1, Wm2, bm2, Wm3, bm3, Wm4, bm4)


# speedup vs baseline: 1.2710x; 1.2710x over previous
import functools
import jax
import jax.numpy as jnp
from jax import lax
from jax.experimental import pallas as pl
from jax.experimental.pallas import tpu as pltpu
from jax.experimental.pallas import tpu_sc as plsc

N = 10000
HID = 64
NUM_GRAPHS = 64
E = 320000

NC = 2
NS = 16
NW = NC * NS
G = 128
NG = 79
EPW = NG * G
EPAD = NW * EPW
NPAD = 10240
ROWS_PER_TILE = NPAD // NS


def _ka_body(x_ref, as_ref, ad_ref, w_ref, h_ref, asrc_ref, adst_ref,
             num0_ref, den0_ref):
    h = jnp.dot(x_ref[...], w_ref[...], preferred_element_type=jnp.float32)
    h_ref[...] = h
    asrc = jnp.sum(h * as_ref[...], axis=1, keepdims=True)
    adst = jnp.sum(h * ad_ref[...], axis=1, keepdims=True)
    asrc_ref[...] = asrc
    adst_ref[...] = adst
    e = asrc + adst
    e = jnp.where(e >= 0, e, 0.2 * e)
    exs = jnp.exp(e)
    num0_ref[...] = h * exs
    den0_ref[...] = exs


def _layer_prologue(x, W, a_s, a_d):
    return pl.pallas_call(
        _ka_body,
        out_shape=(
            jax.ShapeDtypeStruct((N, HID), jnp.float32),
            jax.ShapeDtypeStruct((N, 1), jnp.float32),
            jax.ShapeDtypeStruct((N, 1), jnp.float32),
            jax.ShapeDtypeStruct((N, HID), jnp.float32),
            jax.ShapeDtypeStruct((N, 1), jnp.float32),
        ),
    )(x, a_s.reshape(1, HID), a_d.reshape(1, HID), W)


def _sc_edge_body(h_hbm, asrc_hbm, adst_hbm, src_hbm, dst_hbm, znum_hbm,
                  zden_hbm, num_out, den_out,
                  src_v, dst_v, ex_v, asrc_v, adst_v, rows_v, num_sh, den_sh,
                  sem, sem_sn, sem_sd):
    c = lax.axis_index("c")
    s = lax.axis_index("s")
    wid = s * NC + c

    pltpu.sync_copy(asrc_hbm, asrc_v)
    pltpu.sync_copy(adst_hbm, adst_v)
    pltpu.sync_copy(src_hbm.at[wid], src_v)
    pltpu.sync_copy(dst_hbm.at[wid], dst_v)

    row0 = s * ROWS_PER_TILE
    pltpu.sync_copy(znum_hbm.at[pl.ds(row0, ROWS_PER_TILE)],
                    num_sh.at[pl.ds(row0, ROWS_PER_TILE)])
    pltpu.sync_copy(zden_hbm.at[pl.ds(row0, ROWS_PER_TILE)],
                    den_sh.at[pl.ds(row0, ROWS_PER_TILE)])
    plsc.subcore_barrier()

    def vec_body(k, carry):
        g, b = carry
        base = k * 16
        sidx = src_v[g, pl.ds(base, 16)]
        didx = dst_v[g, pl.ds(base, 16)]
        a = plsc.load_gather(asrc_v, [sidx])
        bb = plsc.load_gather(adst_v, [didx])
        e = a + bb
        e = jnp.where(e >= 0.0, e, 0.2 * e)
        ex16 = jnp.exp(e)
        ex_v[g, pl.ds(base, 16)] = ex16
        for j in range(16):
            exj = ex16[j]
            i = base + j
            for q in range(4):
                sl = pl.ds(q * 16, 16)
                rows_v[b, i, sl] = rows_v[b, i, sl] * exj
        return carry

    pltpu.async_copy(h_hbm.at[src_v.at[0]], rows_v.at[0], sem)

    def group_body(g, _):
        b = lax.rem(g, 4)
        p = lax.rem(g, 2)
        pltpu.make_async_copy(h_hbm.at[src_v.at[g]], rows_v.at[b], sem).wait()

        @pl.when(g >= 2)
        def _():
            ob = lax.rem(g + 2, 4)
            pltpu.make_async_copy(rows_v.at[ob], num_sh.at[dst_v.at[g]],
                                  sem_sn.at[p]).wait()
            pltpu.make_async_copy(ex_v.at[g], den_sh.at[dst_v.at[g]],
                                  sem_sd.at[p]).wait()

        @pl.when(g + 1 < NG)
        def _():
            nb = lax.rem(g + 1, 4)
            pltpu.async_copy(h_hbm.at[src_v.at[g + 1]], rows_v.at[nb], sem)

        plsc.parallel_loop(0, G // 16, carry=(g, b))(vec_body)
        pltpu.async_copy(rows_v.at[b], num_sh.at[dst_v.at[g]], sem_sn.at[p],
                         add=True)
        pltpu.async_copy(ex_v.at[g], den_sh.at[dst_v.at[g]], sem_sd.at[p],
                         add=True)
        return 0

    lax.fori_loop(0, NG, group_body, 0)
    for gg in (NG - 2, NG - 1):
        pltpu.make_async_copy(rows_v.at[gg % 4], num_sh.at[dst_v.at[gg]],
                              sem_sn.at[gg % 2]).wait()
        pltpu.make_async_copy(ex_v.at[gg], den_sh.at[dst_v.at[gg]],
                              sem_sd.at[gg % 2]).wait()
    plsc.subcore_barrier()

    pltpu.sync_copy(num_sh.at[pl.ds(row0, ROWS_PER_TILE)],
                    num_out.at[c, pl.ds(row0, ROWS_PER_TILE)])
    pltpu.sync_copy(den_sh.at[pl.ds(row0, ROWS_PER_TILE)],
                    den_out.at[pl.ds(c * NPAD + row0, ROWS_PER_TILE)])


_sc_edge_pass = pl.kernel(
    _sc_edge_body,
    out_type=(
        jax.ShapeDtypeStruct((NC, NPAD, HID), jnp.float32),
        jax.ShapeDtypeStruct((NC * NPAD,), jnp.float32),
    ),
    mesh=plsc.VectorSubcoreMesh(core_axis_name="c", subcore_axis_name="s"),
    compiler_params=pltpu.CompilerParams(needs_layout_passes=False,
                                         use_tc_tiling_on_sc=False),
    scratch_types=[
        pltpu.VMEM((NG, G), jnp.int32),
        pltpu.VMEM((NG, G), jnp.int32),
        pltpu.VMEM((NG, G), jnp.float32),
        pltpu.VMEM((N,), jnp.float32),
        pltpu.VMEM((N,), jnp.float32),
        pltpu.VMEM((4, G, HID), jnp.float32),
        pltpu.VMEM_SHARED((NPAD, HID), jnp.float32),
        pltpu.VMEM_SHARED((NPAD,), jnp.float32),
        pltpu.SemaphoreType.DMA,
        pltpu.SemaphoreType.DMA((2,)),
        pltpu.SemaphoreType.DMA((2,)),
    ],
)


def _edge_pass(h, asrc, adst, src_r, dst_r, znum, zden):
    num_p, den_p = _sc_edge_pass(
        h, asrc.reshape(N), adst.reshape(N), src_r, dst_r, znum, zden)
    return num_p, den_p


def _kb_body(num0_ref, den0_ref, nump_ref, denp_ref, b_ref, g_ref, be_ref,
             o_ref):
    num = num0_ref[...] + nump_ref[0, :N, :] + nump_ref[1, :N, :]
    den = den0_ref[...] + (denp_ref[pl.ds(0, N)]
                           + denp_ref[pl.ds(NPAD, N)]).reshape(N, 1)
    v = num / (den + 1e-16) + b_ref[...]
    v = jnp.maximum(v, 0.0)
    mu = jnp.mean(v, axis=0, keepdims=True)
    var = jnp.mean((v - mu) ** 2, axis=0, keepdims=True)
    o_ref[...] = g_ref[...] * (v - mu) / jnp.sqrt(var + 1e-5) + be_ref[...]


def _layer_epilogue(num0, den0, num_p, den_p, b, g, be):
    return pl.pallas_call(
        _kb_body,
        out_shape=jax.ShapeDtypeStruct((N, HID), jnp.float32),
    )(num0, den0, num_p, den_p, b.reshape(1, HID),
      g.reshape(1, HID), be.reshape(1, HID))


def _kc_body(h_ref, batch_ref, w1_ref, b1_ref, w2_ref, b2_ref, w3_ref, b3_ref,
             w4_ref, b4_ref, o_ref):
    gids = jax.lax.broadcasted_iota(jnp.int32, (N, NUM_GRAPHS), 1)
    onehot = (batch_ref[...] == gids).astype(jnp.float32)
    pooled = jax.lax.dot_general(
        onehot, h_ref[...], (((0,), (0,)), ((), ())),
        preferred_element_type=jnp.float32,
        precision=jax.lax.Precision.HIGHEST)
    o = jnp.dot(pooled, w1_ref[...], preferred_element_type=jnp.float32) + b1_ref[...]
    o = jnp.dot(o, w2_ref[...], preferred_element_type=jnp.float32) + b2_ref[...]
    o = jnp.dot(o, w3_ref[...], preferred_element_type=jnp.float32) + b3_ref[...]
    o = jnp.dot(o, w4_ref[...], preferred_element_type=jnp.float32) + b4_ref[...]
    o_ref[...] = o


def _head(h, batch, Wm1, bm1, Wm2, bm2, Wm3, bm3, Wm4, bm4):
    return pl.pallas_call(
        _kc_body,
        out_shape=jax.ShapeDtypeStruct((NUM_GRAPHS, 1), jnp.float32),
    )(h, batch.reshape(N, 1), Wm1, bm1.reshape(1, HID), Wm2, bm2.reshape(1, HID),
      Wm3, bm3.reshape(1, HID), Wm4, bm4.reshape(1, 1))


def _kba_body(num0_ref, den0_ref, nump_ref, denp_ref, b_ref, g_ref, be_ref,
              as_ref, ad_ref, w_ref,
              h_ref, asrc_ref, adst_ref, num0n_ref, den0n_ref):
    num = num0_ref[...] + nump_ref[0, :N, :] + nump_ref[1, :N, :]
    den = den0_ref[...] + (denp_ref[pl.ds(0, N)]
                           + denp_ref[pl.ds(NPAD, N)]).reshape(N, 1)
    v = num / (den + 1e-16) + b_ref[...]
    v = jnp.maximum(v, 0.0)
    mu = jnp.mean(v, axis=0, keepdims=True)
    var = jnp.mean((v - mu) ** 2, axis=0, keepdims=True)
    z = g_ref[...] * (v - mu) / jnp.sqrt(var + 1e-5) + be_ref[...]
    h = jnp.dot(z, w_ref[...], preferred_element_type=jnp.float32)
    h_ref[...] = h
    asrc = jnp.sum(h * as_ref[...], axis=1, keepdims=True)
    adst = jnp.sum(h * ad_ref[...], axis=1, keepdims=True)
    asrc_ref[...] = asrc
    adst_ref[...] = adst
    e = asrc + adst
    e = jnp.where(e >= 0, e, 0.2 * e)
    exs = jnp.exp(e)
    num0n_ref[...] = h * exs
    den0n_ref[...] = exs


def _mid_fused(num0, den0, num_p, den_p, b, g, be, W, a_s, a_d):
    return pl.pallas_call(
        _kba_body,
        out_shape=(
            jax.ShapeDtypeStruct((N, HID), jnp.float32),
            jax.ShapeDtypeStruct((N, 1), jnp.float32),
            jax.ShapeDtypeStruct((N, 1), jnp.float32),
            jax.ShapeDtypeStruct((N, HID), jnp.float32),
            jax.ShapeDtypeStruct((N, 1), jnp.float32),
        ),
    )(num0, den0, num_p, den_p, b.reshape(1, HID), g.reshape(1, HID),
      be.reshape(1, HID), a_s.reshape(1, HID), a_d.reshape(1, HID), W)


def _kbc_body(num0_ref, den0_ref, nump_ref, denp_ref, b_ref, g_ref, be_ref,
              batch_ref, w1_ref, b1_ref, w2_ref, b2_ref, w3_ref, b3_ref,
              w4_ref, b4_ref, o_ref):
    num = num0_ref[...] + nump_ref[0, :N, :] + nump_ref[1, :N, :]
    den = den0_ref[...] + (denp_ref[pl.ds(0, N)]
                           + denp_ref[pl.ds(NPAD, N)]).reshape(N, 1)
    v = num / (den + 1e-16) + b_ref[...]
    v = jnp.maximum(v, 0.0)
    mu = jnp.mean(v, axis=0, keepdims=True)
    var = jnp.mean((v - mu) ** 2, axis=0, keepdims=True)
    h = g_ref[...] * (v - mu) / jnp.sqrt(var + 1e-5) + be_ref[...]
    gids = jax.lax.broadcasted_iota(jnp.int32, (N, NUM_GRAPHS), 1)
    onehot = (batch_ref[...] == gids).astype(jnp.float32)
    pooled = jax.lax.dot_general(
        onehot, h, (((0,), (0,)), ((), ())),
        preferred_element_type=jnp.float32,
        precision=jax.lax.Precision.HIGHEST)
    o = jnp.dot(pooled, w1_ref[...], preferred_element_type=jnp.float32) + b1_ref[...]
    o = jnp.dot(o, w2_ref[...], preferred_element_type=jnp.float32) + b2_ref[...]
    o = jnp.dot(o, w3_ref[...], preferred_element_type=jnp.float32) + b3_ref[...]
    o = jnp.dot(o, w4_ref[...], preferred_element_type=jnp.float32) + b4_ref[...]
    o_ref[...] = o


def _tail_fused(num0, den0, num_p, den_p, b, g, be, batch,
                Wm1, bm1, Wm2, bm2, Wm3, bm3, Wm4, bm4):
    return pl.pallas_call(
        _kbc_body,
        out_shape=jax.ShapeDtypeStruct((NUM_GRAPHS, 1), jnp.float32),
    )(num0, den0, num_p, den_p, b.reshape(1, HID), g.reshape(1, HID),
      be.reshape(1, HID), batch.reshape(N, 1), Wm1, bm1.reshape(1, HID),
      Wm2, bm2.reshape(1, HID), Wm3, bm3.reshape(1, HID), Wm4,
      bm4.reshape(1, 1))


@jax.jit
def kernel(x, edge_index, batch, W1, a_src1, a_dst1, b1, g1, be1,
           W2, a_src2, a_dst2, b2, g2, be2,
           Wm1, bm1, Wm2, bm2, Wm3, bm3, Wm4, bm4):
    npad_e = EPAD - E
    src_r = jnp.concatenate(
        [edge_index[0], jnp.zeros((npad_e,), jnp.int32)]).reshape(NW, NG, G)
    dst_r = jnp.concatenate(
        [edge_index[1], jnp.full((npad_e,), N, jnp.int32)]).reshape(NW, NG, G)
    znum = jnp.zeros((NPAD, HID), jnp.float32)
    zden = jnp.zeros((NPAD,), jnp.float32)

    h1, asrc1, adst1, num01, den01 = _layer_prologue(x, W1, a_src1, a_dst1)
    num_p1, den_p1 = _edge_pass(h1, asrc1, adst1, src_r, dst_r, znum, zden)
    h2, asrc2, adst2, num02, den02 = _mid_fused(
        num01, den01, num_p1, den_p1, b1, g1, be1, W2, a_src2, a_dst2)
    num_p2, den_p2 = _edge_pass(h2, asrc2, adst2, src_r, dst_r, znum, zden)
    return _tail_fused(num02, den02, num_p2, den_p2, b2, g2, be2, batch,
                       Wm1, bm1, Wm2, bm2, Wm3, bm3, Wm4, bm4)
